# Initial kernel scaffold; baseline (speedup 1.0000x reference)
import jax
import jax.numpy as jnp
from jax.experimental import pallas as pl


def kernel(rec_current, visited_time, pattern):
    raise NotImplementedError("write your pallas kernel here")


# speedup vs baseline: 2.9200x; 2.9200x over previous
import functools

import jax
import jax.numpy as jnp
from jax import lax
from jax.experimental import pallas as pl
from jax.experimental.pallas import tpu as pltpu
from jax.experimental.pallas import tpu_sc as plsc

_LANES = 16


def _build_gather(n_rows, n_pos, d, n_workers, rows_per_stream):
    assert n_rows % (n_workers * rows_per_stream) == 0
    rows_per_worker = n_rows // n_workers
    streams_per_worker = rows_per_worker // rows_per_stream

    mesh = plsc.VectorSubcoreMesh(core_axis_name="c", subcore_axis_name="s")

    @functools.partial(
        pl.kernel,
        mesh=mesh,
        out_type=jax.ShapeDtypeStruct((n_rows, d), jnp.float32),
        scratch_types=[
            pltpu.VMEM((rows_per_worker,), jnp.int32),
            pltpu.VMEM((rows_per_stream, d), jnp.float32),
            pltpu.SemaphoreType.DMA,
            pltpu.SemaphoreType.DMA,
        ],
    )
    def gather_kernel(idx_hbm, pattern_hbm, out_hbm, idx_v, rows_v, gsem, ssem):
        num_cores = lax.axis_size("c")
        wid = lax.axis_index("s") * num_cores + lax.axis_index("c")

        pltpu.sync_copy(idx_hbm.at[pl.ds(wid * rows_per_worker, rows_per_worker)], idx_v)

        def mod_body(r, carry):
            sl = pl.ds(r * _LANES, _LANES)
            idx_v[sl] = lax.rem(idx_v[sl], n_pos)
            return carry

        lax.fori_loop(0, rows_per_worker // _LANES, mod_body, 0)

        out_base = wid * rows_per_worker

        def stream_body(j, carry):
            idx_sl = idx_v.at[pl.ds(j * rows_per_stream, rows_per_stream)]
            pltpu.async_copy(pattern_hbm.at[idx_sl], rows_v, gsem).wait()
            pltpu.async_copy(
                rows_v, out_hbm.at[pl.ds(out_base + j * rows_per_stream, rows_per_stream)], ssem
            ).wait()
            return carry

        lax.fori_loop(0, streams_per_worker, stream_body, 0)

    return gather_kernel


def kernel(rec_current, visited_time, pattern):
    b, s = rec_current.shape
    n_pos, d = pattern.shape
    n_rows = b * s
    gather = _build_gather(n_rows, n_pos, d, n_workers=32, rows_per_stream=128)
    out = gather(visited_time.reshape(n_rows), pattern)
    return out.reshape(b, s, d)


# speedup vs baseline: 2.9436x; 1.0081x over previous
import functools

import jax
import jax.numpy as jnp
from jax import lax
from jax.experimental import pallas as pl
from jax.experimental.pallas import tpu as pltpu
from jax.experimental.pallas import tpu_sc as plsc

_LANES = 16


def _build_gather(n_rows, n_pos, d, n_workers, rows_per_stream):
    assert n_rows % (n_workers * rows_per_stream) == 0
    rows_per_worker = n_rows // n_workers
    streams_per_worker = rows_per_worker // rows_per_stream

    mesh = plsc.VectorSubcoreMesh(core_axis_name="c", subcore_axis_name="s")

    @functools.partial(
        pl.kernel,
        mesh=mesh,
        out_type=jax.ShapeDtypeStruct((n_rows, d), jnp.float32),
        scratch_types=[
            pltpu.VMEM((rows_per_worker,), jnp.int32),
            pltpu.VMEM((rows_per_stream, d), jnp.float32),
            pltpu.VMEM((rows_per_stream, d), jnp.float32),
            pltpu.SemaphoreType.DMA,
            pltpu.SemaphoreType.DMA,
            pltpu.SemaphoreType.DMA,
            pltpu.SemaphoreType.DMA,
        ],
    )
    def gather_kernel(
        idx_hbm, pattern_hbm, out_hbm, idx_v, rows0, rows1, gsem0, gsem1, ssem0, ssem1
    ):
        num_cores = lax.axis_size("c")
        wid = lax.axis_index("s") * num_cores + lax.axis_index("c")

        pltpu.sync_copy(idx_hbm.at[pl.ds(wid * rows_per_worker, rows_per_worker)], idx_v)

        def mod_body(r, carry):
            sl = pl.ds(r * _LANES, _LANES)
            idx_v[sl] = lax.rem(idx_v[sl], n_pos)
            return carry

        lax.fori_loop(0, rows_per_worker // _LANES, mod_body, 0)

        out_base = wid * rows_per_worker
        bufs = ((rows0, gsem0, ssem0), (rows1, gsem1, ssem1))
        n_streams = streams_per_worker

        def idx_slice(j):
            return idx_v.at[pl.ds(j * rows_per_stream, rows_per_stream)]

        def gather_copy(j, rows, gsem):
            return pltpu.make_async_copy(pattern_hbm.at[idx_slice(j)], rows, gsem)

        gather_copy(0, rows0, gsem0).start()

        def group_body(g, carry):
            for p in range(2):
                rows, gsem, ssem = bufs[p]
                j = g * 2 + p
                gather_copy(j, rows, gsem).wait()
                scatter = pltpu.make_async_copy(
                    rows,
                    out_hbm.at[pl.ds(out_base + j * rows_per_stream, rows_per_stream)],
                    ssem,
                )
                scatter.start()
                nxt, ngsem, _ = bufs[p ^ 1]

                @pl.when(j + 1 < n_streams)
                def _():
                    gather_copy(j + 1, nxt, ngsem).start()

                scatter.wait()
            return carry

        lax.fori_loop(0, n_streams // 2, group_body, 0)

    return gather_kernel


def kernel(rec_current, visited_time, pattern):
    b, s = rec_current.shape
    n_pos, d = pattern.shape
    n_rows = b * s
    gather = _build_gather(n_rows, n_pos, d, n_workers=32, rows_per_stream=128)
    out = gather(visited_time.reshape(n_rows), pattern)
    return out.reshape(b, s, d)


# speedup vs baseline: 2.9601x; 1.0056x over previous
import functools

import jax
import jax.numpy as jnp
from jax import lax
from jax.experimental import pallas as pl
from jax.experimental.pallas import tpu as pltpu
from jax.experimental.pallas import tpu_sc as plsc

_LANES = 16
_NBUF = 5
_GLEAD = 2


def _build_gather(n_rows, n_pos, d, n_workers, rows_per_stream):
    assert n_rows % (n_workers * rows_per_stream) == 0
    rows_per_worker = n_rows // n_workers
    streams_per_worker = rows_per_worker // rows_per_stream

    mesh = plsc.VectorSubcoreMesh(core_axis_name="c", subcore_axis_name="s")

    @functools.partial(
        pl.kernel,
        mesh=mesh,
        out_type=jax.ShapeDtypeStruct((n_rows, d), jnp.float32),
        scratch_types=[
            pltpu.VMEM((rows_per_worker,), jnp.int32),
        ]
        + [pltpu.VMEM((rows_per_stream, d), jnp.float32) for _ in range(_NBUF)]
        + [pltpu.SemaphoreType.DMA for _ in range(2 * _NBUF)],
    )
    def gather_kernel(idx_hbm, pattern_hbm, out_hbm, idx_v, *bufs_and_sems):
        rows = bufs_and_sems[:_NBUF]
        gsem = bufs_and_sems[_NBUF : 2 * _NBUF]
        ssem = bufs_and_sems[2 * _NBUF : 3 * _NBUF]
        num_cores = lax.axis_size("c")
        wid = lax.axis_index("s") * num_cores + lax.axis_index("c")

        pltpu.sync_copy(idx_hbm.at[pl.ds(wid * rows_per_worker, rows_per_worker)], idx_v)

        def mod_body(r, carry):
            sl = pl.ds(r * _LANES, _LANES)
            idx_v[sl] = lax.rem(idx_v[sl], n_pos)
            return carry

        lax.fori_loop(0, rows_per_worker // _LANES, mod_body, 0)

        out_base = wid * rows_per_worker
        n_streams = streams_per_worker
        assert n_streams % _NBUF == 0 and n_streams >= 2 * _NBUF

        def gather_copy(j, b):
            idx_sl = idx_v.at[pl.ds(j * rows_per_stream, rows_per_stream)]
            return pltpu.make_async_copy(pattern_hbm.at[idx_sl], rows[b], gsem[b])

        def scatter_copy(j, b):
            dst = out_hbm.at[pl.ds(out_base + j * rows_per_stream, rows_per_stream)]
            return pltpu.make_async_copy(rows[b], dst, ssem[b])

        gather_copy(0, 0).start()
        gather_copy(1, 1).start()

        def group_body(g, carry):
            for p in range(_NBUF):
                b = p
                j = g * _NBUF + p
                gather_copy(j, b).wait()
                scatter_copy(j, b).start()
                b_next = (p + _GLEAD) % _NBUF

                @pl.when(j >= _NBUF - _GLEAD)
                def _():
                    scatter_copy(j - (_NBUF - _GLEAD), b_next).wait()

                @pl.when(j + _GLEAD < n_streams)
                def _():
                    gather_copy(j + _GLEAD, b_next).start()
            return carry

        lax.fori_loop(0, n_streams // _NBUF, group_body, 0)

        for t in range(n_streams - (_NBUF - _GLEAD), n_streams):
            scatter_copy(t, t % _NBUF).wait()

    return gather_kernel


def kernel(rec_current, visited_time, pattern):
    b, s = rec_current.shape
    n_pos, d = pattern.shape
    n_rows = b * s
    gather = _build_gather(n_rows, n_pos, d, n_workers=32, rows_per_stream=128)
    out = gather(visited_time.reshape(n_rows), pattern)
    return out.reshape(b, s, d)
